# Initial kernel scaffold; baseline (speedup 1.0000x reference)
import jax
import jax.numpy as jnp
from jax.experimental import pallas as pl


def kernel(x_user, x_item, edge_index_user_to_item, edge_index_item_to_user, params):
    raise NotImplementedError("write your pallas kernel here")


# speedup vs baseline: 3.7718x; 3.7718x over previous
import functools

import jax
import jax.numpy as jnp
from jax import lax
from jax.experimental import pallas as pl
from jax.experimental.pallas import tpu as pltpu
from jax.experimental.pallas import tpu_sc as plsc

N = 10000
NP = 10240
H = 128
E = 320000
NC = 2
NS = 16
NW = NC * NS
EPW = E // NW
CH = 80
NCH = EPW // CH
RPT = NP // NS
CW = 128
CWT = 8

@functools.cache
def _make_seg_sum():
    mesh = plsc.VectorSubcoreMesh(core_axis_name="c", subcore_axis_name="s",
                                  num_cores=NC, num_subcores=NS)
    return pl.kernel(
        _seg_sum_body,
        out_type=jax.ShapeDtypeStruct((NC, NP, H), jnp.float32),
        mesh=mesh,
        scratch_types=[
            pltpu.VMEM((CH,), jnp.int32),
            pltpu.VMEM((CH,), jnp.int32),
            pltpu.VMEM((CH, H), jnp.float32),
            pltpu.VMEM_SHARED((NP, H), jnp.float32),
            pltpu.SemaphoreType.DMA,
        ],
    )


def _seg_sum(h, src, dst):
    return _make_seg_sum()(h, src, dst)


def _seg_sum_body(h_hbm, src_hbm, dst_hbm, out_hbm, sidx, didx, rows, acc, sem):
    c = lax.axis_index("c")
    s = lax.axis_index("s")
    wid = s * NC + c

    zero = jnp.zeros((16,), jnp.float32)

    def zrow(i, carry):
        for j in range(H // 16):
            rows[i, pl.ds(j * 16, 16)] = zero
        return carry

    lax.fori_loop(0, CH, zrow, 0)
    for t in range(RPT // CH):
        pltpu.sync_copy(rows, acc.at[pl.ds(s * RPT + t * CH, CH)])
    plsc.subcore_barrier()

    ebase = wid * EPW

    def body(k, carry):
        b = ebase + k * CH
        pltpu.sync_copy(src_hbm.at[pl.ds(b, CH)], sidx)
        pltpu.sync_copy(dst_hbm.at[pl.ds(b, CH)], didx)
        pltpu.async_copy(h_hbm.at[sidx], rows, sem).wait()
        pltpu.sync_copy(rows, acc.at[didx], add=True)
        return carry

    lax.fori_loop(0, NCH, body, 0)
    plsc.subcore_barrier()
    pltpu.sync_copy(acc.at[pl.ds(s * RPT, RPT)], out_hbm.at[c, pl.ds(s * RPT, RPT)])


@functools.cache
def _make_seg_count():
    mesh = plsc.VectorSubcoreMesh(core_axis_name="c", subcore_axis_name="s",
                                  num_cores=NC, num_subcores=NS)
    return pl.kernel(
        _seg_count_body,
        out_type=jax.ShapeDtypeStruct((NC, NP, CW), jnp.float32),
        mesh=mesh,
        scratch_types=[
            pltpu.VMEM((CH,), jnp.int32),
            pltpu.VMEM((CH, CW), jnp.float32),
            pltpu.VMEM_SHARED((NP, CW), jnp.float32),
        ],
    )


def _seg_count(dst):
    ones = jnp.ones((CH, CW), jnp.float32)
    zeros = jnp.zeros((RPT, CW), jnp.float32)
    return _make_seg_count()(dst, ones, zeros)


def _seg_count_body(dst_hbm, ones_hbm, zeros_hbm, out_hbm, didx, ones, acc):
    c = lax.axis_index("c")
    s = lax.axis_index("s")
    wid = s * NC + c

    pltpu.sync_copy(ones_hbm, ones)
    pltpu.sync_copy(zeros_hbm, acc.at[pl.ds(s * RPT, RPT)])
    plsc.subcore_barrier()

    ebase = wid * EPW

    def body(k, carry):
        b = ebase + k * CH
        pltpu.sync_copy(dst_hbm.at[pl.ds(b, CH)], didx)
        pltpu.sync_copy(ones, acc.at[didx], add=True)
        return carry

    lax.fori_loop(0, NCH, body, 0)
    plsc.subcore_barrier()
    pltpu.sync_copy(acc.at[pl.ds(s * RPT, RPT)], out_hbm.at[c, pl.ds(s * RPT, RPT)])


_BR = 1280


def _dot(a, b):
    return jnp.dot(a, b, preferred_element_type=jnp.float32,
                   precision=lax.Precision.HIGHEST)


def _in_proj_body(x_ref, a_ref, c_ref, o_ref):
    o_ref[...] = jnp.maximum(_dot(x_ref[...], a_ref[...]) + c_ref[...], 0.0)


def _in_proj(x, a, cvec):
    grid = NP // _BR
    return pl.pallas_call(
        _in_proj_body,
        grid=(grid,),
        in_specs=[
            pl.BlockSpec((_BR, H), lambda i: (i, 0)),
            pl.BlockSpec((H, H), lambda i: (0, 0)),
            pl.BlockSpec((1, H), lambda i: (0, 0)),
        ],
        out_specs=pl.BlockSpec((_BR, H), lambda i: (i, 0)),
        out_shape=jax.ShapeDtypeStruct((NP, H), jnp.float32),
    )(x, a, cvec)


def _layer_body(a0_ref, a1_ref, cnt_ref, h_ref, al_ref, ar_ref, c_ref, o_ref):
    inv = 1.0 / jnp.maximum(cnt_ref[...][:, :1], 1.0)
    mean = (a0_ref[...] + a1_ref[...]) * inv
    h = h_ref[...]
    z = _dot(mean, al_ref[...]) + _dot(h, ar_ref[...]) + c_ref[...]
    o_ref[...] = jnp.maximum(z, 0.0) + h


def _layer_update(agg, cnt, h, al, ar, cvec):
    grid = NP // _BR
    return pl.pallas_call(
        _layer_body,
        grid=(grid,),
        in_specs=[
            pl.BlockSpec((_BR, H), lambda i: (i, 0)),
            pl.BlockSpec((_BR, H), lambda i: (i, 0)),
            pl.BlockSpec((_BR, CWT), lambda i: (i, 0)),
            pl.BlockSpec((_BR, H), lambda i: (i, 0)),
            pl.BlockSpec((H, H), lambda i: (0, 0)),
            pl.BlockSpec((H, H), lambda i: (0, 0)),
            pl.BlockSpec((1, H), lambda i: (0, 0)),
        ],
        out_specs=pl.BlockSpec((_BR, H), lambda i: (i, 0)),
        out_shape=jax.ShapeDtypeStruct((NP, H), jnp.float32),
    )(agg[0], agg[1], cnt, h, al, ar, cvec)


def _final_body(h_ref, a_ref, c_ref, o_ref):
    o_ref[...] = _dot(h_ref[...], a_ref[...]) + c_ref[...]


def _final_proj(h, a, cvec):
    grid = NP // _BR
    return pl.pallas_call(
        _final_body,
        grid=(grid,),
        in_specs=[
            pl.BlockSpec((_BR, H), lambda i: (i, 0)),
            pl.BlockSpec((H, H), lambda i: (0, 0)),
            pl.BlockSpec((1, H), lambda i: (0, 0)),
        ],
        out_specs=pl.BlockSpec((_BR, H), lambda i: (i, 0)),
        out_shape=jax.ShapeDtypeStruct((NP, H), jnp.float32),
    )(h, a, cvec)


_BN_S = 1.0 / jnp.sqrt(jnp.float32(1.0 + 1e-5))


def kernel(x_user, x_item, edge_index_user_to_item, edge_index_item_to_user,
           params):
    src_ui = edge_index_user_to_item[0]
    dst_ui = edge_index_user_to_item[1]
    src_iu = edge_index_item_to_user[0]
    dst_iu = edge_index_item_to_user[1]

    pad = ((0, NP - N), (0, 0))
    xs = {"user": jnp.pad(x_user, pad), "item": jnp.pad(x_item, pad)}

    cnt_item = _seg_count(dst_ui)
    cnt_user = _seg_count(dst_iu)
    cnt_item = (cnt_item[0] + cnt_item[1])[:, :CWT]
    cnt_user = (cnt_user[0] + cnt_user[1])[:, :CWT]
    cnt = {"item": cnt_item, "user": cnt_user}

    h = {}
    for nt in ("user", "item"):
        W, b = params["lin_in"][nt]
        w2, b2 = params["bn_in"][nt]
        s = w2 * _BN_S
        a = W.T * s[None, :]
        cvec = (b * s + b2)[None, :]
        h[nt] = _in_proj(xs[nt], a, cvec)

    for layer in params["layers"]:
        agg_item = _seg_sum(h["user"], src_ui, dst_ui)
        agg_user = _seg_sum(h["item"], src_iu, dst_iu)
        new_h = {}
        for nt, agg, conv_key in (("item", agg_item, "user_to_item"),
                                  ("user", agg_user, "item_to_user")):
            Wl, bl, Wr = layer["conv"][conv_key]
            w2, b2 = layer["bn"][nt]
            s = w2 * _BN_S
            al = Wl.T * s[None, :]
            ar = Wr.T * s[None, :]
            cvec = (bl * s + b2)[None, :]
            new_h[nt] = _layer_update(agg, cnt[nt], h[nt], al, ar, cvec)
        h = new_h

    W, b = params["final"]
    out_user = _final_proj(h["user"], W.T, b[None, :])
    out_item = _final_proj(h["item"], W.T, b[None, :])
    return (out_user[:N], out_item[:N])


# speedup vs baseline: 8.6400x; 2.2907x over previous
import functools

import jax
import jax.numpy as jnp
from jax import lax
from jax.experimental import pallas as pl
from jax.experimental.pallas import tpu as pltpu
from jax.experimental.pallas import tpu_sc as plsc

N = 10000
NP = 10240
H = 128
E = 320000
NC = 2
NS = 16
NW = NC * NS
EPW = E // NW
CH = 80
NCH = EPW // CH
RPT = NP // NS
CW = 128
CWT = 8

@functools.cache
def _make_seg_sum():
    mesh = plsc.VectorSubcoreMesh(core_axis_name="c", subcore_axis_name="s",
                                  num_cores=NC, num_subcores=NS)
    return pl.kernel(
        _seg_sum_body,
        out_type=jax.ShapeDtypeStruct((NC, NP, H), jnp.float32),
        mesh=mesh,
        scratch_types=[
            pltpu.VMEM((EPW,), jnp.int32),
            pltpu.VMEM((EPW,), jnp.int32),
            pltpu.VMEM((CH,), jnp.int32),
            pltpu.VMEM((CH,), jnp.int32),
            pltpu.VMEM((CH, H), jnp.float32),
            pltpu.VMEM((CH, H), jnp.float32),
            pltpu.VMEM_SHARED((NP, H), jnp.float32),
            pltpu.SemaphoreType.DMA,
        ],
    )


def _seg_sum(h, src, dst):
    return _make_seg_sum()(h, src, dst)


def _seg_sum_body(h_hbm, src_hbm, dst_hbm, out_hbm, sidx, didx, dc0, dc1,
                  rows0, rows1, acc, sem):
    c = lax.axis_index("c")
    s = lax.axis_index("s")
    wid = s * NC + c

    pltpu.sync_copy(src_hbm.at[pl.ds(wid * EPW, EPW)], sidx)
    pltpu.sync_copy(dst_hbm.at[pl.ds(wid * EPW, EPW)], didx)

    zero = jnp.zeros((16,), jnp.float32)

    def zrow(i, carry):
        for j in range(H // 16):
            rows0[i, pl.ds(j * 16, 16)] = zero
        return carry

    lax.fori_loop(0, CH, zrow, 0)
    for t in range(RPT // CH):
        pltpu.sync_copy(rows0, acc.at[pl.ds(s * RPT + t * CH, CH)])
    plsc.subcore_barrier()

    def copy_idx(ch, dc):
        for j in range(CH // 16):
            dc[pl.ds(j * 16, 16)] = didx[pl.ds(ch * CH + j * 16, 16)]

    def fire(ch, buf, dc):
        copy_idx(ch, dc)
        pltpu.async_copy(h_hbm.at[sidx.at[pl.ds(ch * CH, CH)]], buf, sem)

    def wait(buf):
        pltpu.make_async_copy(h_hbm.at[sidx.at[pl.ds(0, CH)]], buf, sem).wait()

    def scat(buf, dc):
        pltpu.sync_copy(buf, acc.at[dc], add=True)

    fire(0, rows0, dc0)

    def body(i, carry):
        ch = 2 * i
        fire(ch + 1, rows1, dc1)
        wait(rows0)
        scat(rows0, dc0)
        fire(ch + 2, rows0, dc0)
        wait(rows1)
        scat(rows1, dc1)
        return carry

    lax.fori_loop(0, (NCH - 1) // 2, body, 0)
    wait(rows0)
    scat(rows0, dc0)

    plsc.subcore_barrier()
    pltpu.sync_copy(acc.at[pl.ds(s * RPT, RPT)], out_hbm.at[c, pl.ds(s * RPT, RPT)])


@functools.cache
def _make_seg_count():
    mesh = plsc.VectorSubcoreMesh(core_axis_name="c", subcore_axis_name="s",
                                  num_cores=NC, num_subcores=NS)
    return pl.kernel(
        _seg_count_body,
        out_type=jax.ShapeDtypeStruct((NC, NP, CW), jnp.float32),
        mesh=mesh,
        scratch_types=[
            pltpu.VMEM((EPW,), jnp.int32),
            pltpu.VMEM((CH,), jnp.int32),
            pltpu.VMEM((CH,), jnp.int32),
            pltpu.VMEM((CH, CW), jnp.float32),
            pltpu.VMEM_SHARED((NP, CW), jnp.float32),
            pltpu.SemaphoreType.DMA,
        ],
    )


def _seg_count(dst):
    ones = jnp.ones((CH, CW), jnp.float32)
    zeros = jnp.zeros((RPT, CW), jnp.float32)
    return _make_seg_count()(dst, ones, zeros)


def _seg_count_body(dst_hbm, ones_hbm, zeros_hbm, out_hbm, didx, dc0, dc1,
                    ones, acc, sem):
    c = lax.axis_index("c")
    s = lax.axis_index("s")
    wid = s * NC + c

    pltpu.sync_copy(dst_hbm.at[pl.ds(wid * EPW, EPW)], didx)
    pltpu.sync_copy(ones_hbm, ones)
    pltpu.sync_copy(zeros_hbm, acc.at[pl.ds(s * RPT, RPT)])
    plsc.subcore_barrier()

    def stage(ch, dc):
        for j in range(CH // 16):
            dc[pl.ds(j * 16, 16)] = didx[pl.ds(ch * CH + j * 16, 16)]

    def fire(dc):
        pltpu.async_copy(ones, acc.at[dc], sem, add=True)

    def wait_one(dc):
        pltpu.make_async_copy(ones, acc.at[dc], sem).wait()

    stage(0, dc0)
    fire(dc0)

    def body(i, carry):
        ch = 2 * i
        stage(ch + 1, dc1)
        fire(dc1)
        wait_one(dc0)
        stage(ch + 2, dc0)
        fire(dc0)
        wait_one(dc1)
        return carry

    lax.fori_loop(0, (NCH - 1) // 2, body, 0)
    wait_one(dc0)
    plsc.subcore_barrier()
    pltpu.sync_copy(acc.at[pl.ds(s * RPT, RPT)], out_hbm.at[c, pl.ds(s * RPT, RPT)])


_BR = 1280


def _dot(a, b):
    return jnp.dot(a, b, preferred_element_type=jnp.float32,
                   precision=lax.Precision.HIGHEST)


def _in_proj_body(x_ref, a_ref, c_ref, o_ref):
    o_ref[...] = jnp.maximum(_dot(x_ref[...], a_ref[...]) + c_ref[...], 0.0)


def _in_proj(x, a, cvec):
    grid = NP // _BR
    return pl.pallas_call(
        _in_proj_body,
        grid=(grid,),
        in_specs=[
            pl.BlockSpec((_BR, H), lambda i: (i, 0)),
            pl.BlockSpec((H, H), lambda i: (0, 0)),
            pl.BlockSpec((1, H), lambda i: (0, 0)),
        ],
        out_specs=pl.BlockSpec((_BR, H), lambda i: (i, 0)),
        out_shape=jax.ShapeDtypeStruct((NP, H), jnp.float32),
    )(x, a, cvec)


def _layer_body(a0_ref, a1_ref, cnt_ref, h_ref, al_ref, ar_ref, c_ref, o_ref):
    inv = 1.0 / jnp.maximum(cnt_ref[...][:, :1], 1.0)
    mean = (a0_ref[...] + a1_ref[...]) * inv
    h = h_ref[...]
    z = _dot(mean, al_ref[...]) + _dot(h, ar_ref[...]) + c_ref[...]
    o_ref[...] = jnp.maximum(z, 0.0) + h


def _layer_update(agg, cnt, h, al, ar, cvec):
    grid = NP // _BR
    return pl.pallas_call(
        _layer_body,
        grid=(grid,),
        in_specs=[
            pl.BlockSpec((_BR, H), lambda i: (i, 0)),
            pl.BlockSpec((_BR, H), lambda i: (i, 0)),
            pl.BlockSpec((_BR, CWT), lambda i: (i, 0)),
            pl.BlockSpec((_BR, H), lambda i: (i, 0)),
            pl.BlockSpec((H, H), lambda i: (0, 0)),
            pl.BlockSpec((H, H), lambda i: (0, 0)),
            pl.BlockSpec((1, H), lambda i: (0, 0)),
        ],
        out_specs=pl.BlockSpec((_BR, H), lambda i: (i, 0)),
        out_shape=jax.ShapeDtypeStruct((NP, H), jnp.float32),
    )(agg[0], agg[1], cnt, h, al, ar, cvec)


def _final_body(h_ref, a_ref, c_ref, o_ref):
    o_ref[...] = _dot(h_ref[...], a_ref[...]) + c_ref[...]


def _final_proj(h, a, cvec):
    grid = NP // _BR
    return pl.pallas_call(
        _final_body,
        grid=(grid,),
        in_specs=[
            pl.BlockSpec((_BR, H), lambda i: (i, 0)),
            pl.BlockSpec((H, H), lambda i: (0, 0)),
            pl.BlockSpec((1, H), lambda i: (0, 0)),
        ],
        out_specs=pl.BlockSpec((_BR, H), lambda i: (i, 0)),
        out_shape=jax.ShapeDtypeStruct((NP, H), jnp.float32),
    )(h, a, cvec)


_BN_S = 1.0 / jnp.sqrt(jnp.float32(1.0 + 1e-5))


def kernel(x_user, x_item, edge_index_user_to_item, edge_index_item_to_user,
           params):
    src_ui = edge_index_user_to_item[0]
    dst_ui = edge_index_user_to_item[1]
    src_iu = edge_index_item_to_user[0]
    dst_iu = edge_index_item_to_user[1]

    pad = ((0, NP - N), (0, 0))
    xs = {"user": jnp.pad(x_user, pad), "item": jnp.pad(x_item, pad)}

    cnt_item = _seg_count(dst_ui)
    cnt_user = _seg_count(dst_iu)
    cnt_item = (cnt_item[0] + cnt_item[1])[:, :CWT]
    cnt_user = (cnt_user[0] + cnt_user[1])[:, :CWT]
    cnt = {"item": cnt_item, "user": cnt_user}

    h = {}
    for nt in ("user", "item"):
        W, b = params["lin_in"][nt]
        w2, b2 = params["bn_in"][nt]
        s = w2 * _BN_S
        a = W.T * s[None, :]
        cvec = (b * s + b2)[None, :]
        h[nt] = _in_proj(xs[nt], a, cvec)

    for layer in params["layers"]:
        agg_item = _seg_sum(h["user"], src_ui, dst_ui)
        agg_user = _seg_sum(h["item"], src_iu, dst_iu)
        new_h = {}
        for nt, agg, conv_key in (("item", agg_item, "user_to_item"),
                                  ("user", agg_user, "item_to_user")):
            Wl, bl, Wr = layer["conv"][conv_key]
            w2, b2 = layer["bn"][nt]
            s = w2 * _BN_S
            al = Wl.T * s[None, :]
            ar = Wr.T * s[None, :]
            cvec = (bl * s + b2)[None, :]
            new_h[nt] = _layer_update(agg, cnt[nt], h[nt], al, ar, cvec)
        h = new_h

    W, b = params["final"]
    out_user = _final_proj(h["user"], W.T, b[None, :])
    out_item = _final_proj(h["item"], W.T, b[None, :])
    return (out_user[:N], out_item[:N])


# speedup vs baseline: 9.8958x; 1.1453x over previous
import functools

import jax
import jax.numpy as jnp
from jax import lax
from jax.experimental import pallas as pl
from jax.experimental.pallas import tpu as pltpu
from jax.experimental.pallas import tpu_sc as plsc

N = 10000
NP = 10240
H = 128
E = 320000
NC = 2
NS = 16
NW = NC * NS
EPW = E // NW
CH = 80
NCH = EPW // CH
RPT = NP // NS
CW = 128
CWT = 8

@functools.cache
def _make_seg_sum():
    mesh = plsc.VectorSubcoreMesh(core_axis_name="c", subcore_axis_name="s",
                                  num_cores=NC, num_subcores=NS)
    return pl.kernel(
        _seg_sum_body,
        out_type=jax.ShapeDtypeStruct((NC, NP, H), jnp.float32),
        mesh=mesh,
        scratch_types=[
            pltpu.VMEM((CH,), jnp.int32),
            pltpu.VMEM((CH,), jnp.int32),
            pltpu.VMEM((CH,), jnp.int32),
            pltpu.VMEM((CH,), jnp.int32),
            pltpu.VMEM((CH,), jnp.int32),
            pltpu.VMEM((CH,), jnp.int32),
            pltpu.VMEM((CH,), jnp.int32),
            pltpu.VMEM((CH,), jnp.int32),
            pltpu.VMEM((CH, H), jnp.float32),
            pltpu.VMEM((CH, H), jnp.float32),
            pltpu.VMEM((CH, H), jnp.float32),
            pltpu.VMEM((CH, H), jnp.float32),
            pltpu.VMEM_SHARED((NP, H), jnp.float32),
            pltpu.SemaphoreType.DMA,
            pltpu.SemaphoreType.DMA,
        ],
    )


def _seg_sum(h, src, dst):
    return _make_seg_sum()(h, src, dst)


def _seg_sum_body(h_hbm, src_hbm, dst_hbm, out_hbm, si0, si1, si2, si3,
                  dc0, dc1, dc2, dc3, rows0, rows1, rows2, rows3, acc,
                  gsem, isem):
    sis = (si0, si1, si2, si3)
    dcs = (dc0, dc1, dc2, dc3)
    rows = (rows0, rows1, rows2, rows3)
    c = lax.axis_index("c")
    s = lax.axis_index("s")
    wid = s * NC + c
    ebase = wid * EPW

    zero = jnp.zeros((16,), jnp.float32)

    def zrow(i, carry):
        for j in range(H // 16):
            rows0[i, pl.ds(j * 16, 16)] = zero
        return carry

    lax.fori_loop(0, CH, zrow, 0)
    for t in range(RPT // CH):
        pltpu.sync_copy(rows0, acc.at[pl.ds(s * RPT + t * CH, CH)])
    plsc.subcore_barrier()

    def fire_idx(ch, b):
        pltpu.async_copy(src_hbm.at[pl.ds(ebase + ch * CH, CH)], sis[b], isem)
        pltpu.async_copy(dst_hbm.at[pl.ds(ebase + ch * CH, CH)], dcs[b], isem)

    def wait_idx(b):
        pltpu.make_async_copy(src_hbm.at[pl.ds(0, CH)], sis[b], isem).wait()
        pltpu.make_async_copy(dst_hbm.at[pl.ds(0, CH)], dcs[b], isem).wait()

    def fire_gather(b):
        pltpu.async_copy(h_hbm.at[sis[b]], rows[b], gsem)

    def wait_gather(b):
        pltpu.make_async_copy(h_hbm.at[sis[0]], rows[b], gsem).wait()

    def scat(b):
        pltpu.sync_copy(rows[b], acc.at[dcs[b]], add=True)

    fire_idx(0, 0)
    fire_idx(1, 1)
    fire_idx(2, 2)
    wait_idx(0)
    fire_gather(0)
    wait_idx(1)
    fire_gather(1)

    def slot(ch, b):
        wait_idx((b + 2) % 4)
        fire_gather((b + 2) % 4)
        wait_gather(b)
        fire_idx(ch + 3, (b + 3) % 4)
        scat(b)

    def body(i, carry):
        ch = 4 * i
        for b in range(4):
            slot(ch + b, b)
        return carry

    K = (NCH - 5) // 4
    lax.fori_loop(0, K, body, 0)
    base = 4 * K
    slot(base, 0)
    slot(base + 1, 1)
    wait_idx(0)
    fire_gather(0)
    wait_gather(2)
    scat(2)
    wait_gather(3)
    scat(3)
    wait_gather(0)
    scat(0)

    plsc.subcore_barrier()
    pltpu.sync_copy(acc.at[pl.ds(s * RPT, RPT)], out_hbm.at[c, pl.ds(s * RPT, RPT)])


@functools.cache
def _make_seg_count():
    mesh = plsc.VectorSubcoreMesh(core_axis_name="c", subcore_axis_name="s",
                                  num_cores=NC, num_subcores=NS)
    return pl.kernel(
        _seg_count_body,
        out_type=jax.ShapeDtypeStruct((NC, NP, CW), jnp.float32),
        mesh=mesh,
        scratch_types=[
            pltpu.VMEM((EPW,), jnp.int32),
            pltpu.VMEM((CH,), jnp.int32),
            pltpu.VMEM((CH,), jnp.int32),
            pltpu.VMEM((CH, CW), jnp.float32),
            pltpu.VMEM_SHARED((NP, CW), jnp.float32),
            pltpu.SemaphoreType.DMA,
        ],
    )


def _seg_count(dst):
    ones = jnp.ones((CH, CW), jnp.float32)
    zeros = jnp.zeros((RPT, CW), jnp.float32)
    return _make_seg_count()(dst, ones, zeros)


def _seg_count_body(dst_hbm, ones_hbm, zeros_hbm, out_hbm, didx, dc0, dc1,
                    ones, acc, sem):
    c = lax.axis_index("c")
    s = lax.axis_index("s")
    wid = s * NC + c

    pltpu.sync_copy(dst_hbm.at[pl.ds(wid * EPW, EPW)], didx)
    pltpu.sync_copy(ones_hbm, ones)
    pltpu.sync_copy(zeros_hbm, acc.at[pl.ds(s * RPT, RPT)])
    plsc.subcore_barrier()

    def stage(ch, dc):
        for j in range(CH // 16):
            dc[pl.ds(j * 16, 16)] = didx[pl.ds(ch * CH + j * 16, 16)]

    def fire(dc):
        pltpu.async_copy(ones, acc.at[dc], sem, add=True)

    def wait_one(dc):
        pltpu.make_async_copy(ones, acc.at[dc], sem).wait()

    stage(0, dc0)
    fire(dc0)

    def body(i, carry):
        ch = 2 * i
        stage(ch + 1, dc1)
        fire(dc1)
        wait_one(dc0)
        stage(ch + 2, dc0)
        fire(dc0)
        wait_one(dc1)
        return carry

    lax.fori_loop(0, (NCH - 1) // 2, body, 0)
    wait_one(dc0)
    plsc.subcore_barrier()
    pltpu.sync_copy(acc.at[pl.ds(s * RPT, RPT)], out_hbm.at[c, pl.ds(s * RPT, RPT)])


_BR = 1280


def _dot(a, b):
    return jnp.dot(a, b, preferred_element_type=jnp.float32,
                   precision=lax.Precision.HIGHEST)


def _in_proj_body(x_ref, a_ref, c_ref, o_ref):
    o_ref[...] = jnp.maximum(_dot(x_ref[...], a_ref[...]) + c_ref[...], 0.0)


def _in_proj(x, a, cvec):
    grid = NP // _BR
    return pl.pallas_call(
        _in_proj_body,
        grid=(grid,),
        in_specs=[
            pl.BlockSpec((_BR, H), lambda i: (i, 0)),
            pl.BlockSpec((H, H), lambda i: (0, 0)),
            pl.BlockSpec((1, H), lambda i: (0, 0)),
        ],
        out_specs=pl.BlockSpec((_BR, H), lambda i: (i, 0)),
        out_shape=jax.ShapeDtypeStruct((NP, H), jnp.float32),
    )(x, a, cvec)


def _layer_body(a0_ref, a1_ref, cnt_ref, h_ref, al_ref, ar_ref, c_ref, o_ref):
    inv = 1.0 / jnp.maximum(cnt_ref[...][:, :1], 1.0)
    mean = (a0_ref[...] + a1_ref[...]) * inv
    h = h_ref[...]
    z = _dot(mean, al_ref[...]) + _dot(h, ar_ref[...]) + c_ref[...]
    o_ref[...] = jnp.maximum(z, 0.0) + h


def _layer_update(agg, cnt, h, al, ar, cvec):
    grid = NP // _BR
    return pl.pallas_call(
        _layer_body,
        grid=(grid,),
        in_specs=[
            pl.BlockSpec((_BR, H), lambda i: (i, 0)),
            pl.BlockSpec((_BR, H), lambda i: (i, 0)),
            pl.BlockSpec((_BR, CWT), lambda i: (i, 0)),
            pl.BlockSpec((_BR, H), lambda i: (i, 0)),
            pl.BlockSpec((H, H), lambda i: (0, 0)),
            pl.BlockSpec((H, H), lambda i: (0, 0)),
            pl.BlockSpec((1, H), lambda i: (0, 0)),
        ],
        out_specs=pl.BlockSpec((_BR, H), lambda i: (i, 0)),
        out_shape=jax.ShapeDtypeStruct((NP, H), jnp.float32),
    )(agg[0], agg[1], cnt, h, al, ar, cvec)


def _final_body(h_ref, a_ref, c_ref, o_ref):
    o_ref[...] = _dot(h_ref[...], a_ref[...]) + c_ref[...]


def _final_proj(h, a, cvec):
    grid = NP // _BR
    return pl.pallas_call(
        _final_body,
        grid=(grid,),
        in_specs=[
            pl.BlockSpec((_BR, H), lambda i: (i, 0)),
            pl.BlockSpec((H, H), lambda i: (0, 0)),
            pl.BlockSpec((1, H), lambda i: (0, 0)),
        ],
        out_specs=pl.BlockSpec((_BR, H), lambda i: (i, 0)),
        out_shape=jax.ShapeDtypeStruct((NP, H), jnp.float32),
    )(h, a, cvec)


_BN_S = 1.0 / jnp.sqrt(jnp.float32(1.0 + 1e-5))


def kernel(x_user, x_item, edge_index_user_to_item, edge_index_item_to_user,
           params):
    src_ui = edge_index_user_to_item[0]
    dst_ui = edge_index_user_to_item[1]
    src_iu = edge_index_item_to_user[0]
    dst_iu = edge_index_item_to_user[1]

    pad = ((0, NP - N), (0, 0))
    xs = {"user": jnp.pad(x_user, pad), "item": jnp.pad(x_item, pad)}

    cnt_item = _seg_count(dst_ui)
    cnt_user = _seg_count(dst_iu)
    cnt_item = (cnt_item[0] + cnt_item[1])[:, :CWT]
    cnt_user = (cnt_user[0] + cnt_user[1])[:, :CWT]
    cnt = {"item": cnt_item, "user": cnt_user}

    h = {}
    for nt in ("user", "item"):
        W, b = params["lin_in"][nt]
        w2, b2 = params["bn_in"][nt]
        s = w2 * _BN_S
        a = W.T * s[None, :]
        cvec = (b * s + b2)[None, :]
        h[nt] = _in_proj(xs[nt], a, cvec)

    for layer in params["layers"]:
        agg_item = _seg_sum(h["user"], src_ui, dst_ui)
        agg_user = _seg_sum(h["item"], src_iu, dst_iu)
        new_h = {}
        for nt, agg, conv_key in (("item", agg_item, "user_to_item"),
                                  ("user", agg_user, "item_to_user")):
            Wl, bl, Wr = layer["conv"][conv_key]
            w2, b2 = layer["bn"][nt]
            s = w2 * _BN_S
            al = Wl.T * s[None, :]
            ar = Wr.T * s[None, :]
            cvec = (bl * s + b2)[None, :]
            new_h[nt] = _layer_update(agg, cnt[nt], h[nt], al, ar, cvec)
        h = new_h

    W, b = params["final"]
    out_user = _final_proj(h["user"], W.T, b[None, :])
    out_item = _final_proj(h["item"], W.T, b[None, :])
    return (out_user[:N], out_item[:N])
